# Initial kernel scaffold; baseline (speedup 1.0000x reference)
import jax
import jax.numpy as jnp
from jax.experimental import pallas as pl


def kernel(x, W1, b1, Wg, bg, We, be, Wr, br, Wc, bc, W2, b2):
    raise NotImplementedError("write your pallas kernel here")


# speedup vs baseline: 14.8167x; 14.8167x over previous
import functools

import jax
import jax.numpy as jnp
from jax import lax
from jax.experimental import pallas as pl
from jax.experimental.pallas import tpu as pltpu
from jax.experimental.pallas import tpu_sc as plsc

L = 16

_WE_O = 0
_BE_O = 128
_SB = 160
_W1_O = 160
_B1_O = 224
_WG_O = 240
_BG_O = 368
_WC_O = 376
_BC_O = 408
_WR_O = 410
_BR_O = 426
_B2_O = 430
_W_LEN = 448
_N_SROW = (_W_LEN - _SB) // L


def _rnd_bf16(v):
    u = plsc.bitcast(v, jnp.int32)
    lsb = lax.shift_right_logical(u, 16) & 1
    u = u + jnp.int32(0x7FFF) + lsb
    return plsc.bitcast(u & jnp.int32(-65536), jnp.float32)


@functools.partial(jax.jit, static_argnames=("n_tokens",))
def _sc_moe(xT, wflat, n_tokens):
    mesh = plsc.VectorSubcoreMesh(
        core_axis_name="c", subcore_axis_name="s", num_cores=2, num_subcores=16)
    nw = mesh.num_cores * mesh.num_subcores
    tpw = n_tokens // nw
    assert n_tokens % (nw * L) == 0

    def body(xT_hbm, w_hbm, out_hbm, xv, wv, ov):
        wid = lax.axis_index("s") * mesh.num_cores + lax.axis_index("c")
        base = wid * tpw
        pltpu.sync_copy(w_hbm, wv)
        for d in range(4):
            pltpu.sync_copy(xT_hbm.at[d, pl.ds(base, tpw)], xv.at[d])

        wrow = [wv[pl.ds(_SB + L * k, L)] for k in range(_N_SROW)]

        def sw(i):
            j = i - _SB
            return wrow[j // L][j % L]

        w1 = [[sw(_W1_O + d * 16 + j) for j in range(16)] for d in range(4)]
        b1 = [sw(_B1_O + j) for j in range(16)]
        wg = [[sw(_WG_O + j * 8 + e) for e in range(8)] for j in range(16)]
        bg = [sw(_BG_O + e) for e in range(8)]
        wc = [[sw(_WC_O + j * 2 + c) for c in range(2)] for j in range(16)]
        bc = [sw(_BC_O + c) for c in range(2)]
        wr = [[sw(_WR_O + d * 4 + o) for o in range(4)] for d in range(4)]
        br = [sw(_BR_O + o) for o in range(4)]
        b2 = [sw(_B2_O + o) for o in range(4)]

        def group(g, carry):
            ts = pl.ds(g * L, L)
            xs = [xv[d, ts] for d in range(4)]

            h = [xs[0] * w1[0][j] + xs[1] * w1[1][j] + xs[2] * w1[2][j]
                 + xs[3] * w1[3][j] + b1[j] for j in range(16)]
            hb = [_rnd_bf16(h[j]) for j in range(16)]

            l = []
            for e in range(8):
                acc = hb[0] * wg[0][e]
                for j in range(1, 16):
                    acc = acc + hb[j] * wg[j][e]
                l.append(acc + bg[e])
            m = l[0]
            for e in range(1, 8):
                m = jnp.maximum(m, l[e])
            idx = jnp.full((L,), 0, jnp.int32)
            for e in range(7, -1, -1):
                idx = jnp.where(l[e] == m, jnp.int32(e), idx)
            s = jnp.exp(l[0] - m)
            for e in range(1, 8):
                s = s + jnp.exp(l[e] - m)
            gate = 1.0 / s

            t0 = hb[0] * wc[0][0]
            t1 = hb[0] * wc[0][1]
            for j in range(1, 16):
                t0 = t0 + hb[j] * wc[j][0]
                t1 = t1 + hb[j] * wc[j][1]
            t0 = t0 + bc[0]
            t1 = t1 + bc[1]
            m2 = jnp.maximum(t0, t1)
            e0 = jnp.exp(t0 - m2)
            e1 = jnp.exp(t1 - m2)
            se = e0 + e1
            c1 = e1 / se
            g0 = gate * (e0 / se)

            pe = idx * 16
            pb = idx * 4 + _BE_O
            y = []
            for o in range(4):
                acc = plsc.load_gather(wv, [pb + o])
                for d in range(4):
                    w = plsc.load_gather(wv, [pe + (d * 4 + o)])
                    acc = acc + xs[d] * w
                y.append(acc)

            for o in range(4):
                r = (xs[0] * wr[0][o] + xs[1] * wr[1][o] + xs[2] * wr[2][o]
                     + xs[3] * wr[3][o] + br[o])
                ov[o, ts] = g0 * y[o] + c1 * r + b2[o]
            return carry

        lax.fori_loop(0, tpw // L, group, 0)
        for d in range(4):
            pltpu.sync_copy(ov.at[d], out_hbm.at[d, pl.ds(base, tpw)])

    run = pl.kernel(
        body,
        out_type=jax.ShapeDtypeStruct((4, n_tokens), jnp.float32),
        mesh=mesh,
        scratch_types=[
            pltpu.VMEM((4, tpw), jnp.float32),
            pltpu.VMEM((_W_LEN,), jnp.float32),
            pltpu.VMEM((4, tpw), jnp.float32),
        ],
        compiler_params=pltpu.CompilerParams(needs_layout_passes=False),
    )
    return run(xT, wflat)


def _bf16r(w):
    u = jax.lax.bitcast_convert_type(w, jnp.int32)
    lsb = jax.lax.shift_right_logical(u, 16) & 1
    u = u + jnp.int32(0x7FFF) + lsb
    return jax.lax.bitcast_convert_type(u & jnp.int32(-65536), jnp.float32)


def kernel(x, W1, b1, Wg, bg, We, be, Wr, br, Wc, bc, W2, b2):
    n = x.shape[0]
    We_f = jnp.einsum("di,eij,jo->edo", W1, We, W2)
    be_f = (jnp.einsum("i,eij->ej", b1, We) + be) @ W2
    Wr_f = W1 @ Wr @ W2
    br_f = (b1 @ Wr + br) @ W2
    wflat = jnp.concatenate([
        We_f.reshape(-1), be_f.reshape(-1),
        _bf16r(W1).reshape(-1), b1,
        _bf16r(Wg).reshape(-1), bg,
        _bf16r(Wc).reshape(-1), bc,
        Wr_f.reshape(-1), br_f, b2,
        jnp.zeros((_W_LEN - 434,), jnp.float32),
    ])
    xTr = _bf16r(x.T)
    outT = _sc_moe(xTr, wflat, n)
    return outT.T


# speedup vs baseline: 15.7226x; 1.0611x over previous
import functools

import jax
import jax.numpy as jnp
from jax import lax
from jax.experimental import pallas as pl
from jax.experimental.pallas import tpu as pltpu
from jax.experimental.pallas import tpu_sc as plsc

L = 16

_WE_O = 0
_BE_O = 128
_SB = 160
_W1_O = 160
_B1_O = 224
_WG_O = 240
_BG_O = 368
_WC_O = 376
_BC_O = 408
_WR_O = 410
_BR_O = 426
_B2_O = 430
_W_LEN = 448
_N_SROW = (_W_LEN - _SB) // L


def _rnd_bf16(v):
    u = plsc.bitcast(v, jnp.int32)
    lsb = lax.shift_right_logical(u, 16) & 1
    u = u + jnp.int32(0x7FFF) + lsb
    return plsc.bitcast(u & jnp.int32(-65536), jnp.float32)


@functools.partial(jax.jit, static_argnames=("n_tokens",))
def _sc_moe(xT, wflat, n_tokens):
    mesh = plsc.VectorSubcoreMesh(
        core_axis_name="c", subcore_axis_name="s", num_cores=2, num_subcores=16)
    nw = mesh.num_cores * mesh.num_subcores
    tpw = n_tokens // nw
    assert n_tokens % (nw * L) == 0

    def body(xT_hbm, w_hbm, out_hbm, xv, wv, ov):
        wid = lax.axis_index("s") * mesh.num_cores + lax.axis_index("c")
        base = wid * tpw
        pltpu.sync_copy(w_hbm, wv)
        for d in range(4):
            pltpu.sync_copy(xT_hbm.at[d, pl.ds(base, tpw)], xv.at[d])

        wrow = [wv[pl.ds(_SB + L * k, L)] for k in range(_N_SROW)]

        def sw(i):
            j = i - _SB
            return wrow[j // L][j % L]

        w1 = [[sw(_W1_O + d * 16 + j) for j in range(16)] for d in range(4)]
        b1 = [sw(_B1_O + j) for j in range(16)]
        wg = [[sw(_WG_O + j * 8 + e) for e in range(8)] for j in range(16)]
        bg = [sw(_BG_O + e) for e in range(8)]
        wc = [[sw(_WC_O + j * 2 + c) for c in range(2)] for j in range(16)]
        bc = [sw(_BC_O + c) for c in range(2)]
        wr = [[sw(_WR_O + d * 4 + o) for o in range(4)] for d in range(4)]
        br = [sw(_BR_O + o) for o in range(4)]
        b2 = [sw(_B2_O + o) for o in range(4)]

        @plsc.parallel_loop(0, tpw // L, unroll=2)
        def group(g):
            ts = pl.ds(g * L, L)
            xs = [xv[d, ts] for d in range(4)]

            h = [xs[0] * w1[0][j] + xs[1] * w1[1][j] + xs[2] * w1[2][j]
                 + xs[3] * w1[3][j] + b1[j] for j in range(16)]
            hb = [_rnd_bf16(h[j]) for j in range(16)]

            l = []
            for e in range(8):
                acc = hb[0] * wg[0][e]
                for j in range(1, 16):
                    acc = acc + hb[j] * wg[j][e]
                l.append(acc + bg[e])
            m = l[0]
            for e in range(1, 8):
                m = jnp.maximum(m, l[e])
            idx = jnp.full((L,), 0, jnp.int32)
            for e in range(7, -1, -1):
                idx = jnp.where(l[e] == m, jnp.int32(e), idx)
            s = jnp.exp(l[0] - m)
            for e in range(1, 8):
                s = s + jnp.exp(l[e] - m)
            gate = 1.0 / s

            t0 = hb[0] * wc[0][0]
            t1 = hb[0] * wc[0][1]
            for j in range(1, 16):
                t0 = t0 + hb[j] * wc[j][0]
                t1 = t1 + hb[j] * wc[j][1]
            t0 = t0 + bc[0]
            t1 = t1 + bc[1]
            m2 = jnp.maximum(t0, t1)
            e0 = jnp.exp(t0 - m2)
            e1 = jnp.exp(t1 - m2)
            se = e0 + e1
            c1 = e1 / se
            g0 = gate * (e0 / se)

            pe = idx * 16
            pb = idx * 4 + _BE_O
            y = []
            for o in range(4):
                acc = plsc.load_gather(wv, [pb + o])
                for d in range(4):
                    w = plsc.load_gather(wv, [pe + (d * 4 + o)])
                    acc = acc + xs[d] * w
                y.append(acc)

            for o in range(4):
                r = (xs[0] * wr[0][o] + xs[1] * wr[1][o] + xs[2] * wr[2][o]
                     + xs[3] * wr[3][o] + br[o])
                ov[o, ts] = g0 * y[o] + c1 * r + b2[o]

        for d in range(4):
            pltpu.sync_copy(ov.at[d], out_hbm.at[d, pl.ds(base, tpw)])

    run = pl.kernel(
        body,
        out_type=jax.ShapeDtypeStruct((4, n_tokens), jnp.float32),
        mesh=mesh,
        scratch_types=[
            pltpu.VMEM((4, tpw), jnp.float32),
            pltpu.VMEM((_W_LEN,), jnp.float32),
            pltpu.VMEM((4, tpw), jnp.float32),
        ],
        compiler_params=pltpu.CompilerParams(needs_layout_passes=False),
    )
    return run(xT, wflat)


def _bf16r(w):
    u = jax.lax.bitcast_convert_type(w, jnp.int32)
    lsb = jax.lax.shift_right_logical(u, 16) & 1
    u = u + jnp.int32(0x7FFF) + lsb
    return jax.lax.bitcast_convert_type(u & jnp.int32(-65536), jnp.float32)


def kernel(x, W1, b1, Wg, bg, We, be, Wr, br, Wc, bc, W2, b2):
    n = x.shape[0]
    We_f = jnp.einsum("di,eij,jo->edo", W1, We, W2)
    be_f = (jnp.einsum("i,eij->ej", b1, We) + be) @ W2
    Wr_f = W1 @ Wr @ W2
    br_f = (b1 @ Wr + br) @ W2
    wflat = jnp.concatenate([
        We_f.reshape(-1), be_f.reshape(-1),
        _bf16r(W1).reshape(-1), b1,
        _bf16r(Wg).reshape(-1), bg,
        _bf16r(Wc).reshape(-1), bc,
        Wr_f.reshape(-1), br_f, b2,
        jnp.zeros((_W_LEN - 434,), jnp.float32),
    ])
    xTr = _bf16r(x.T)
    outT = _sc_moe(xTr, wflat, n)
    return outT.T
